# baseline (speedup 1.0000x reference)
import jax
import jax.numpy as jnp

def kernel(node_idx, table):
    B = node_idx.shape[0]
    tpk = jnp.reshape(table, (500000, 128))
    tpk = jax.lax.optimization_barrier(tpk)
    return tpk[:B, :64]


# speedup vs baseline: 1.1131x; 1.1131x over previous
import functools

import jax
import jax.numpy as jnp
from jax import lax
from jax.experimental import pallas as pl
from jax.experimental.pallas import tpu as pltpu
from jax.experimental.pallas import tpu_sc as plsc


def _sc_gather(idx2, tpad, b_per_w, nc, ns, D):
    nw = nc * ns

    mesh = plsc.VectorSubcoreMesh(core_axis_name="c", subcore_axis_name="s")

    @functools.partial(
        pl.kernel,
        mesh=mesh,
        out_type=jax.ShapeDtypeStruct((nw * b_per_w, 2 * D), jnp.float32),
        scratch_types=[
            pltpu.VMEM((b_per_w,), jnp.int32),
            pltpu.VMEM((b_per_w, 2 * D), jnp.float32),
            pltpu.SemaphoreType.DMA,
        ],
    )
    def body(idx_hbm, tpad_hbm, out_hbm, idx_v, rows_v, sem):
        wid = lax.axis_index("s") * nc + lax.axis_index("c")
        base = wid * b_per_w
        pltpu.sync_copy(idx_hbm.at[wid], idx_v)
        cp = pltpu.async_copy(tpad_hbm.at[idx_v], rows_v, sem)
        cp.wait()
        pltpu.sync_copy(rows_v, out_hbm.at[pl.ds(base, b_per_w)])

    return body(idx2, tpad)


def kernel(node_idx, table):
    B = node_idx.shape[0]
    V, D = table.shape
    info = plsc.get_sparse_core_info()
    nc, ns = info.num_cores, info.num_subcores
    nw = nc * ns
    b_per_w = B // nw

    idx2 = node_idx.astype(jnp.int32).reshape(nw, b_per_w)
    tpad = jnp.pad(table, ((0, 0), (0, D)))
    out = _sc_gather(idx2, tpad, b_per_w, nc, ns, D)
    return out[:, :D]
